# Initial kernel scaffold; baseline (speedup 1.0000x reference)
import jax
import jax.numpy as jnp
from jax.experimental import pallas as pl


def kernel(features, edge_index, adj_values, W):
    raise NotImplementedError("write your pallas kernel here")


# speedup vs baseline: 3.6554x; 3.6554x over previous
import functools

import jax
import jax.numpy as jnp
from jax import lax
from jax.experimental import pallas as pl
from jax.experimental.pallas import tpu as pltpu
from jax.experimental.pallas import tpu_sc as plsc

N = 10000
E = 320000
D_IN = 128
D_OUT = 128
DH = D_OUT // 2
K = 128
NC = 2
NS = 16
NWIN = -(-E // K)
NWIN_PAD = -(-NWIN // (2 * NS)) * (2 * NS)
WPT = NWIN_PAD // NS
E_PAD = NWIN_PAD * K
CHUNK_ROWS = 80
NCHUNK_TOT = N // CHUNK_ROWS


def _adj_group(abuf, g):
    return abuf[pl.ds(g * 16, 16)]


def _lane_bcast(vec, j):
    return jnp.full((16,), vec[j])


def _core_ids():
    return lax.axis_index("c"), lax.axis_index("s")


def _scatter_add(rows_buf, acc_sp, ebuf):
    pltpu.sync_copy(rows_buf, acc_sp.at[ebuf.at[0]], add=True)


def _mm_body(x_ref, w_ref, o0_ref, o1_ref):
    s = jnp.dot(x_ref[...], w_ref[...], preferred_element_type=jnp.float32)
    o0_ref[...] = s[:, :DH]
    o1_ref[...] = s[:, DH:]


def _matmul_halves(features, W):
    bm = 1000
    return pl.pallas_call(
        _mm_body,
        grid=(N // bm,),
        in_specs=[
            pl.BlockSpec((bm, D_IN), lambda i: (i, 0)),
            pl.BlockSpec((D_IN, D_OUT), lambda i: (0, 0)),
        ],
        out_specs=[
            pl.BlockSpec((bm, DH), lambda i: (i, 0)),
            pl.BlockSpec((bm, DH), lambda i: (i, 0)),
        ],
        out_shape=[
            jax.ShapeDtypeStruct((N, DH), jnp.float32),
            jax.ShapeDtypeStruct((N, DH), jnp.float32),
        ],
    )(features, W)


def _sc_spmm(support0, support1, edata, adj3d):
    mesh = plsc.VectorSubcoreMesh(
        core_axis_name="c", subcore_axis_name="s", num_cores=NC, num_subcores=NS
    )

    @functools.partial(
        pl.kernel,
        out_type=jax.ShapeDtypeStruct((NC, N, DH), jnp.float32),
        mesh=mesh,
        compiler_params=pltpu.CompilerParams(use_tc_tiling_on_sc=False),
        scratch_types=[
            pltpu.VMEM_SHARED((N, DH), jnp.float32),
            pltpu.VMEM_SHARED((N, DH), jnp.float32),
            pltpu.VMEM((2, K), jnp.int32),
            pltpu.VMEM((2, K), jnp.int32),
            pltpu.VMEM((K,), jnp.float32),
            pltpu.VMEM((K,), jnp.float32),
            pltpu.VMEM((K, DH), jnp.float32),
            pltpu.VMEM((K, DH), jnp.float32),
            pltpu.VMEM((CHUNK_ROWS, DH), jnp.float32),
            pltpu.SemaphoreType.DMA,
            pltpu.SemaphoreType.DMA,
            pltpu.SemaphoreType.DMA,
            pltpu.SemaphoreType.DMA,
            pltpu.SemaphoreType.DMA,
        ],
    )
    def spmm(s0_hbm, s1_hbm, edata_hbm, adj_hbm, out_hbm,
             sup_sp, acc_sp, ebuf0, ebuf1, abuf0, abuf1, rows0, rows1,
             obuf, sem_e0, sem_e1, sem_g0, sem_g1, sem):
        c, s = _core_ids()

        zero = jnp.zeros((16,), jnp.float32)

        def zrow(i, _):
            for cc in range(DH // 16):
                obuf[i, pl.ds(cc * 16, 16)] = zero
            return 0

        lax.fori_loop(0, CHUNK_ROWS, zrow, 0)
        nk = (NCHUNK_TOT - 1 - s) // NS + 1

        def zchunk(k2, _):
            m = s + NS * k2
            base = m * CHUNK_ROWS

            @pl.when(c == 0)
            def _():
                pltpu.sync_copy(s0_hbm.at[pl.ds(base, CHUNK_ROWS)],
                                sup_sp.at[pl.ds(base, CHUNK_ROWS)])

            @pl.when(c == 1)
            def _():
                pltpu.sync_copy(s1_hbm.at[pl.ds(base, CHUNK_ROWS)],
                                sup_sp.at[pl.ds(base, CHUNK_ROWS)])

            pltpu.sync_copy(obuf, acc_sp.at[pl.ds(base, CHUNK_ROWS)])
            return 0

        lax.fori_loop(0, nk, zchunk, 0)
        plsc.subcore_barrier()

        bufs = ((ebuf0, abuf0, rows0, sem_e0, sem_g0),
                (ebuf1, abuf1, rows1, sem_e1, sem_g1))
        lo = s * WPT
        hi = (s + 1) * WPT

        def start_edata(w, b):
            eb, ab, _, se, _ = bufs[b]
            pltpu.async_copy(edata_hbm.at[w], eb, se)
            pltpu.async_copy(adj_hbm.at[w, 0], ab, se)

        def wait_edata(w, b):
            eb, ab, _, se, _ = bufs[b]
            pltpu.make_async_copy(edata_hbm.at[w], eb, se).wait()
            pltpu.make_async_copy(adj_hbm.at[w, 0], ab, se).wait()

        def start_gather(b):
            eb, _, rb, _, sg = bufs[b]
            pltpu.async_copy(sup_sp.at[eb.at[1]], rb, sg)

        def wait_gather(b):
            eb, _, rb, _, sg = bufs[b]
            pltpu.make_async_copy(sup_sp.at[eb.at[1]], rb, sg).wait()

        def process(w, b):
            nb = 1 - b
            eb, ab, rb, _, _ = bufs[b]

            @pl.when(w + 1 < hi)
            def _():
                wait_edata(w + 1, nb)
                start_gather(nb)

            wait_gather(b)

            def grp(g, _):
                adj_v = _adj_group(ab, g)
                for j in range(16):
                    e = g * 16 + j
                    a = _lane_bcast(adj_v, j)
                    for cc in range(DH // 16):
                        rb[e, pl.ds(cc * 16, 16)] = (
                            rb[e, pl.ds(cc * 16, 16)] * a
                        )
                return 0

            lax.fori_loop(0, K // 16, grp, 0)
            _scatter_add(rb, acc_sp, eb)

            @pl.when(w + 2 < hi)
            def _():
                start_edata(w + 2, b)

        start_edata(lo, 0)
        start_edata(lo + 1, 1)
        wait_edata(lo, 0)
        start_gather(0)

        def pair_body(p, _):
            w0 = lo + 2 * p
            process(w0, 0)
            process(w0 + 1, 1)
            return 0

        lax.fori_loop(0, WPT // 2, pair_body, 0)
        plsc.subcore_barrier()

        def chunk(k2, _):
            base = (s + NS * k2) * CHUNK_ROWS
            pltpu.sync_copy(acc_sp.at[pl.ds(base, CHUNK_ROWS)], obuf)

            def rrow(i, _):
                for cc in range(DH // 16):
                    v = obuf[i, pl.ds(cc * 16, 16)]
                    obuf[i, pl.ds(cc * 16, 16)] = jnp.maximum(v, 0.0)
                return 0

            lax.fori_loop(0, CHUNK_ROWS, rrow, 0)
            pltpu.sync_copy(obuf, out_hbm.at[c, pl.ds(base, CHUNK_ROWS)])
            return 0

        lax.fori_loop(0, nk, chunk, 0)

    return spmm(support0, support1, edata, adj3d)


def _pack_edges(edge_index, adj_values):
    pad = E_PAD - E
    row = edge_index[0]
    col = edge_index[1]
    if pad:
        spread = (jnp.arange(pad, dtype=jnp.int32) * 521) % N
        row = jnp.concatenate([row, spread])
        col = jnp.concatenate([col, spread])
        adj_values = jnp.concatenate(
            [adj_values, jnp.zeros((pad,), jnp.float32)]
        )
    packed = jnp.stack(
        [row.reshape(NWIN_PAD, K), col.reshape(NWIN_PAD, K)], axis=1
    )
    return packed, adj_values.reshape(NWIN_PAD, 1, K)


def kernel(features, edge_index, adj_values, W):
    support0, support1 = _matmul_halves(features, W)
    edata, adj3d = _pack_edges(edge_index, adj_values)
    halves = _sc_spmm(support0, support1, edata, adj3d)
    return jnp.concatenate([halves[0], halves[1]], axis=-1)


# speedup vs baseline: 3.7432x; 1.0240x over previous
import functools

import jax
import jax.numpy as jnp
from jax import lax
from jax.experimental import pallas as pl
from jax.experimental.pallas import tpu as pltpu
from jax.experimental.pallas import tpu_sc as plsc

N = 10000
E = 320000
D_IN = 128
D_OUT = 128
DH = D_OUT // 2
K = 128
NC = 2
NS = 16
NWIN = -(-E // K)
NWIN_PAD = -(-NWIN // (2 * NS)) * (2 * NS)
WPT = NWIN_PAD // NS
E_PAD = NWIN_PAD * K
CHUNK_ROWS = 80
NCHUNK_TOT = N // CHUNK_ROWS


def _adj_group(abuf, g):
    return abuf[pl.ds(g * 16, 16)]


def _lane_bcast(vec, j):
    return jnp.full((16,), vec[j])


def _core_ids():
    return lax.axis_index("c"), lax.axis_index("s")


def _scatter_add(rows_buf, acc_sp, ebuf):
    pltpu.sync_copy(rows_buf, acc_sp.at[ebuf.at[0]], add=True)


def _mm_body(x_ref, w_ref, o0_ref, o1_ref):
    s = jnp.dot(x_ref[...], w_ref[...], preferred_element_type=jnp.float32)
    o0_ref[...] = s[:, :DH]
    o1_ref[...] = s[:, DH:]


def _matmul_halves(features, W):
    bm = 1000
    return pl.pallas_call(
        _mm_body,
        grid=(N // bm,),
        in_specs=[
            pl.BlockSpec((bm, D_IN), lambda i: (i, 0)),
            pl.BlockSpec((D_IN, D_OUT), lambda i: (0, 0)),
        ],
        out_specs=[
            pl.BlockSpec((bm, DH), lambda i: (i, 0)),
            pl.BlockSpec((bm, DH), lambda i: (i, 0)),
        ],
        out_shape=[
            jax.ShapeDtypeStruct((N, DH), jnp.float32),
            jax.ShapeDtypeStruct((N, DH), jnp.float32),
        ],
    )(features, W)


def _sc_spmm(support0, support1, edata, adj3d):
    mesh = plsc.VectorSubcoreMesh(
        core_axis_name="c", subcore_axis_name="s", num_cores=NC, num_subcores=NS
    )

    @functools.partial(
        pl.kernel,
        out_type=jax.ShapeDtypeStruct((NC, N, DH), jnp.float32),
        mesh=mesh,
        compiler_params=pltpu.CompilerParams(use_tc_tiling_on_sc=False),
        scratch_types=[
            pltpu.VMEM_SHARED((N, DH), jnp.float32),
            pltpu.VMEM((2, K), jnp.int32),
            pltpu.VMEM((2, K), jnp.int32),
            pltpu.VMEM((K,), jnp.float32),
            pltpu.VMEM((K,), jnp.float32),
            pltpu.VMEM((K, DH), jnp.float32),
            pltpu.VMEM((K, DH), jnp.float32),
            pltpu.VMEM((CHUNK_ROWS, DH), jnp.float32),
            pltpu.SemaphoreType.DMA,
            pltpu.SemaphoreType.DMA,
            pltpu.SemaphoreType.DMA,
            pltpu.SemaphoreType.DMA,
            pltpu.SemaphoreType.DMA,
        ],
    )
    def spmm(s0_hbm, s1_hbm, edata_hbm, adj_hbm, out_hbm,
             acc_sp, ebuf0, ebuf1, abuf0, abuf1, rows0, rows1,
             obuf, sem_e0, sem_e1, sem_g0, sem_g1, sem):
        c, s = _core_ids()

        zero = jnp.zeros((16,), jnp.float32)

        def zrow(i, _):
            for cc in range(DH // 16):
                obuf[i, pl.ds(cc * 16, 16)] = zero
            return 0

        lax.fori_loop(0, CHUNK_ROWS, zrow, 0)
        nk = (NCHUNK_TOT - 1 - s) // NS + 1

        def zchunk(k2, _):
            m = s + NS * k2
            base = m * CHUNK_ROWS
            pltpu.sync_copy(obuf, acc_sp.at[pl.ds(base, CHUNK_ROWS)])
            return 0

        lax.fori_loop(0, nk, zchunk, 0)
        plsc.subcore_barrier()

        bufs = ((ebuf0, abuf0, rows0, sem_e0, sem_g0),
                (ebuf1, abuf1, rows1, sem_e1, sem_g1))
        lo = s * WPT
        hi = (s + 1) * WPT

        def start_edata(w, b):
            eb, ab, _, se, _ = bufs[b]
            pltpu.async_copy(edata_hbm.at[w], eb, se)
            pltpu.async_copy(adj_hbm.at[w, 0], ab, se)

        def wait_edata(w, b):
            eb, ab, _, se, _ = bufs[b]
            pltpu.make_async_copy(edata_hbm.at[w], eb, se).wait()
            pltpu.make_async_copy(adj_hbm.at[w, 0], ab, se).wait()

        def start_gather(b):
            eb, _, rb, _, sg = bufs[b]

            @pl.when(c == 0)
            def _():
                pltpu.async_copy(s0_hbm.at[eb.at[1]], rb, sg)

            @pl.when(c == 1)
            def _():
                pltpu.async_copy(s1_hbm.at[eb.at[1]], rb, sg)

        def wait_gather(b):
            eb, _, rb, _, sg = bufs[b]
            pltpu.make_async_copy(s0_hbm.at[eb.at[1]], rb, sg).wait()

        def process(w, b):
            nb = 1 - b
            eb, ab, rb, _, _ = bufs[b]

            @pl.when(w + 1 < hi)
            def _():
                wait_edata(w + 1, nb)
                start_gather(nb)

            wait_gather(b)

            def grp(g, _):
                adj_v = _adj_group(ab, g)
                for j in range(16):
                    e = g * 16 + j
                    a = _lane_bcast(adj_v, j)
                    for cc in range(DH // 16):
                        rb[e, pl.ds(cc * 16, 16)] = (
                            rb[e, pl.ds(cc * 16, 16)] * a
                        )
                return 0

            lax.fori_loop(0, K // 16, grp, 0)
            _scatter_add(rb, acc_sp, eb)

            @pl.when(w + 2 < hi)
            def _():
                start_edata(w + 2, b)

        start_edata(lo, 0)
        start_edata(lo + 1, 1)
        wait_edata(lo, 0)
        start_gather(0)

        def pair_body(p, _):
            w0 = lo + 2 * p
            process(w0, 0)
            process(w0 + 1, 1)
            return 0

        lax.fori_loop(0, WPT // 2, pair_body, 0)
        plsc.subcore_barrier()

        def chunk(k2, _):
            base = (s + NS * k2) * CHUNK_ROWS
            pltpu.sync_copy(acc_sp.at[pl.ds(base, CHUNK_ROWS)], obuf)

            def rrow(i, _):
                for cc in range(DH // 16):
                    v = obuf[i, pl.ds(cc * 16, 16)]
                    obuf[i, pl.ds(cc * 16, 16)] = jnp.maximum(v, 0.0)
                return 0

            lax.fori_loop(0, CHUNK_ROWS, rrow, 0)
            pltpu.sync_copy(obuf, out_hbm.at[c, pl.ds(base, CHUNK_ROWS)])
            return 0

        lax.fori_loop(0, nk, chunk, 0)

    return spmm(support0, support1, edata, adj3d)


def _pack_edges(edge_index, adj_values):
    pad = E_PAD - E
    row = edge_index[0]
    col = edge_index[1]
    if pad:
        spread = (jnp.arange(pad, dtype=jnp.int32) * 521) % N
        row = jnp.concatenate([row, spread])
        col = jnp.concatenate([col, spread])
        adj_values = jnp.concatenate(
            [adj_values, jnp.zeros((pad,), jnp.float32)]
        )
    packed = jnp.stack(
        [row.reshape(NWIN_PAD, K), col.reshape(NWIN_PAD, K)], axis=1
    )
    return packed, adj_values.reshape(NWIN_PAD, 1, K)


def kernel(features, edge_index, adj_values, W):
    support0, support1 = _matmul_halves(features, W)
    edata, adj3d = _pack_edges(edge_index, adj_values)
    halves = _sc_spmm(support0, support1, edata, adj3d)
    return jnp.concatenate([halves[0], halves[1]], axis=-1)


# speedup vs baseline: 4.1763x; 1.1157x over previous
import functools

import jax
import jax.numpy as jnp
from jax import lax
from jax.experimental import pallas as pl
from jax.experimental.pallas import tpu as pltpu
from jax.experimental.pallas import tpu_sc as plsc

N = 10000
E = 320000
D_IN = 128
D_OUT = 128
DH = D_OUT // 2
K = 512
NC = 2
NS = 16
NWIN = -(-E // K)
NWIN_PAD = -(-NWIN // (2 * NS)) * (2 * NS)
WPT = NWIN_PAD // NS
E_PAD = NWIN_PAD * K
CHUNK_ROWS = 80
NCHUNK_TOT = N // CHUNK_ROWS


def _adj_group(abuf, g):
    return abuf[pl.ds(g * 16, 16)]


def _lane_bcast(vec, j):
    return jnp.full((16,), vec[j])


def _core_ids():
    return lax.axis_index("c"), lax.axis_index("s")


def _scatter_add(rows_buf, acc_sp, ebuf):
    pltpu.sync_copy(rows_buf, acc_sp.at[ebuf.at[0]], add=True)


def _mm_body(x_ref, w_ref, o0_ref, o1_ref):
    s = jnp.dot(x_ref[...], w_ref[...], preferred_element_type=jnp.float32)
    o0_ref[...] = s[:, :DH]
    o1_ref[...] = s[:, DH:]


def _matmul_halves(features, W):
    bm = 1000
    return pl.pallas_call(
        _mm_body,
        grid=(N // bm,),
        in_specs=[
            pl.BlockSpec((bm, D_IN), lambda i: (i, 0)),
            pl.BlockSpec((D_IN, D_OUT), lambda i: (0, 0)),
        ],
        out_specs=[
            pl.BlockSpec((bm, DH), lambda i: (i, 0)),
            pl.BlockSpec((bm, DH), lambda i: (i, 0)),
        ],
        out_shape=[
            jax.ShapeDtypeStruct((N, DH), jnp.float32),
            jax.ShapeDtypeStruct((N, DH), jnp.float32),
        ],
    )(features, W)


def _sc_spmm(support0, support1, edata, adj3d):
    mesh = plsc.VectorSubcoreMesh(
        core_axis_name="c", subcore_axis_name="s", num_cores=NC, num_subcores=NS
    )

    @functools.partial(
        pl.kernel,
        out_type=jax.ShapeDtypeStruct((NC, N, DH), jnp.float32),
        mesh=mesh,
        compiler_params=pltpu.CompilerParams(use_tc_tiling_on_sc=False),
        scratch_types=[
            pltpu.VMEM_SHARED((N, DH), jnp.float32),
            pltpu.VMEM((2, K), jnp.int32),
            pltpu.VMEM((2, K), jnp.int32),
            pltpu.VMEM((K,), jnp.float32),
            pltpu.VMEM((K,), jnp.float32),
            pltpu.VMEM((K, DH), jnp.float32),
            pltpu.VMEM((K, DH), jnp.float32),
            pltpu.VMEM((CHUNK_ROWS, DH), jnp.float32),
            pltpu.SemaphoreType.DMA,
            pltpu.SemaphoreType.DMA,
            pltpu.SemaphoreType.DMA,
            pltpu.SemaphoreType.DMA,
            pltpu.SemaphoreType.DMA,
        ],
    )
    def spmm(s0_hbm, s1_hbm, edata_hbm, adj_hbm, out_hbm,
             acc_sp, ebuf0, ebuf1, abuf0, abuf1, rows0, rows1,
             obuf, sem_e0, sem_e1, sem_g0, sem_g1, sem):
        c, s = _core_ids()

        zero = jnp.zeros((16,), jnp.float32)

        def zrow(i, _):
            for cc in range(DH // 16):
                obuf[i, pl.ds(cc * 16, 16)] = zero
            return 0

        lax.fori_loop(0, CHUNK_ROWS, zrow, 0)
        nk = (NCHUNK_TOT - 1 - s) // NS + 1

        def zchunk(k2, _):
            m = s + NS * k2
            base = m * CHUNK_ROWS
            pltpu.sync_copy(obuf, acc_sp.at[pl.ds(base, CHUNK_ROWS)])
            return 0

        lax.fori_loop(0, nk, zchunk, 0)
        plsc.subcore_barrier()

        bufs = ((ebuf0, abuf0, rows0, sem_e0, sem_g0),
                (ebuf1, abuf1, rows1, sem_e1, sem_g1))
        lo = s * WPT
        hi = (s + 1) * WPT

        def start_edata(w, b):
            eb, ab, _, se, _ = bufs[b]
            pltpu.async_copy(edata_hbm.at[w], eb, se)
            pltpu.async_copy(adj_hbm.at[w, 0], ab, se)

        def wait_edata(w, b):
            eb, ab, _, se, _ = bufs[b]
            pltpu.make_async_copy(edata_hbm.at[w], eb, se).wait()
            pltpu.make_async_copy(adj_hbm.at[w, 0], ab, se).wait()

        def start_gather(b):
            eb, _, rb, _, sg = bufs[b]

            @pl.when(c == 0)
            def _():
                pltpu.async_copy(s0_hbm.at[eb.at[1]], rb, sg)

            @pl.when(c == 1)
            def _():
                pltpu.async_copy(s1_hbm.at[eb.at[1]], rb, sg)

        def wait_gather(b):
            eb, _, rb, _, sg = bufs[b]
            pltpu.make_async_copy(s0_hbm.at[eb.at[1]], rb, sg).wait()

        def process(w, b):
            nb = 1 - b
            eb, ab, rb, _, _ = bufs[b]

            @pl.when(w + 1 < hi)
            def _():
                wait_edata(w + 1, nb)
                start_gather(nb)

            wait_gather(b)

            def grp(g, _):
                adj_v = _adj_group(ab, g)
                for j in range(16):
                    e = g * 16 + j
                    a = _lane_bcast(adj_v, j)
                    for cc in range(DH // 16):
                        rb[e, pl.ds(cc * 16, 16)] = (
                            rb[e, pl.ds(cc * 16, 16)] * a
                        )
                return 0

            lax.fori_loop(0, K // 16, grp, 0)
            _scatter_add(rb, acc_sp, eb)

            @pl.when(w + 2 < hi)
            def _():
                start_edata(w + 2, b)

        start_edata(lo, 0)
        start_edata(lo + 1, 1)
        wait_edata(lo, 0)
        start_gather(0)

        def pair_body(p, _):
            w0 = lo + 2 * p
            process(w0, 0)
            process(w0 + 1, 1)
            return 0

        lax.fori_loop(0, WPT // 2, pair_body, 0)
        plsc.subcore_barrier()

        def chunk(k2, _):
            base = (s + NS * k2) * CHUNK_ROWS
            pltpu.sync_copy(acc_sp.at[pl.ds(base, CHUNK_ROWS)], obuf)

            def rrow(i, _):
                for cc in range(DH // 16):
                    v = obuf[i, pl.ds(cc * 16, 16)]
                    obuf[i, pl.ds(cc * 16, 16)] = jnp.maximum(v, 0.0)
                return 0

            lax.fori_loop(0, CHUNK_ROWS, rrow, 0)
            pltpu.sync_copy(obuf, out_hbm.at[c, pl.ds(base, CHUNK_ROWS)])
            return 0

        lax.fori_loop(0, nk, chunk, 0)

    return spmm(support0, support1, edata, adj3d)


def _pack_edges(edge_index, adj_values):
    pad = E_PAD - E
    row = edge_index[0]
    col = edge_index[1]
    if pad:
        spread = (jnp.arange(pad, dtype=jnp.int32) * 521) % N
        row = jnp.concatenate([row, spread])
        col = jnp.concatenate([col, spread])
        adj_values = jnp.concatenate(
            [adj_values, jnp.zeros((pad,), jnp.float32)]
        )
    packed = jnp.stack(
        [row.reshape(NWIN_PAD, K), col.reshape(NWIN_PAD, K)], axis=1
    )
    return packed, adj_values.reshape(NWIN_PAD, 1, K)


def kernel(features, edge_index, adj_values, W):
    support0, support1 = _matmul_halves(features, W)
    edata, adj3d = _pack_edges(edge_index, adj_values)
    halves = _sc_spmm(support0, support1, edata, adj3d)
    return jnp.concatenate([halves[0], halves[1]], axis=-1)
